# Initial kernel scaffold; baseline (speedup 1.0000x reference)
import jax
import jax.numpy as jnp
from jax import lax
from jax.experimental import pallas as pl
from jax.experimental.pallas import tpu as pltpu
from jax.experimental.pallas import tpu_sc as plsc

N = 16384
D = 256
C = 1000
CPAD = 1024
NC = 2
NS = 16
NW = NC * NS
ROWS_PER_TILE = N // NW
CHUNK = 64
NCHUNKS = ROWS_PER_TILE // CHUNK
LANES = 16
DV = D // LANES


def _rsqrt16(t):
    ti = lax.bitcast_convert_type(t, jnp.int32)
    yi = jnp.int32(0x5F3759DF) - lax.shift_right_arithmetic(ti, 1)
    y = lax.bitcast_convert_type(yi, jnp.float32)
    half_t = jnp.float32(0.5) * t
    for _ in range(3):
        y = y * (jnp.float32(1.5) - half_t * y * y)
    return y


def _phase_a_body(features, labels, sums_out, counts_out,
                  featbuf, onesbuf, labelbuf, accum, cntacc):
    c = lax.axis_index("c")
    s = lax.axis_index("s")
    wid = s * NC + c

    zeros16 = jnp.zeros((LANES,), jnp.float32)
    ones16 = jnp.ones((LANES,), jnp.float32)

    def _zrow(r, carry):
        for j in range(DV):
            featbuf[r, pl.ds(j * LANES, LANES)] = zeros16
        onesbuf[r, :] = zeros16
        return carry
    lax.fori_loop(0, CHUNK, _zrow, 0)
    pltpu.sync_copy(featbuf, accum.at[pl.ds(s * CHUNK, CHUNK)])
    pltpu.sync_copy(onesbuf, cntacc.at[pl.ds(s * CHUNK, CHUNK)])

    def _orow(r, carry):
        onesbuf[r, :] = ones16
        return carry
    lax.fori_loop(0, CHUNK, _orow, 0)
    plsc.subcore_barrier()

    base = wid * ROWS_PER_TILE
    for g in range(NCHUNKS):
        row0 = base + g * CHUNK
        pltpu.sync_copy(features.at[pl.ds(row0, CHUNK)], featbuf)
        pltpu.sync_copy(labels.at[pl.ds(row0, CHUNK)], labelbuf)

        def _nrow(r, carry):
            acc = zeros16
            for j in range(DV):
                v = featbuf[r, pl.ds(j * LANES, LANES)]
                acc = acc + v * v
            tot = jnp.maximum(jnp.sum(acc), jnp.float32(1e-35))
            w = _rsqrt16(jnp.broadcast_to(tot, (LANES,)))
            for j in range(DV):
                featbuf[r, pl.ds(j * LANES, LANES)] = (
                    featbuf[r, pl.ds(j * LANES, LANES)] * w)
            return carry
        lax.fori_loop(0, CHUNK, _nrow, 0)

        pltpu.sync_copy(featbuf, accum.at[labelbuf], add=True)
        pltpu.sync_copy(onesbuf, cntacc.at[labelbuf], add=True)

    plsc.subcore_barrier()
    out0 = c * CPAD + s * CHUNK
    pltpu.sync_copy(accum.at[pl.ds(s * CHUNK, CHUNK)],
                    sums_out.at[pl.ds(out0, CHUNK)])
    pltpu.sync_copy(cntacc.at[pl.ds(s * CHUNK, CHUNK)],
                    counts_out.at[pl.ds(out0, CHUNK)])


_phase_a = pl.kernel(
    _phase_a_body,
    out_type=[
        jax.ShapeDtypeStruct((NC * CPAD, D), jnp.float32),
        jax.ShapeDtypeStruct((NC * CPAD, LANES), jnp.float32),
    ],
    mesh=plsc.VectorSubcoreMesh(
        core_axis_name="c", subcore_axis_name="s",
        num_cores=NC, num_subcores=NS),
    scratch_types=[
        pltpu.VMEM((CHUNK, D), jnp.float32),
        pltpu.VMEM((CHUNK, LANES), jnp.float32),
        pltpu.VMEM((CHUNK,), jnp.int32),
        pltpu.VMEM_SHARED((CPAD, D), jnp.float32),
        pltpu.VMEM_SHARED((CPAD, LANES), jnp.float32),
    ],
)


def _phase_b_body(sums_ref, counts_ref, protos_ref, init_ref, out_ref):
    s = sums_ref[0:CPAD, :] + sums_ref[CPAD:2 * CPAD, :]
    cnt = counts_ref[0:CPAD, 0:1] + counts_ref[CPAD:2 * CPAD, 0:1]
    mean = s / jnp.maximum(cnt, jnp.float32(1.0))
    protos = protos_ref[...]
    ema = jnp.float32(0.99) * protos + jnp.float32(0.01) * mean
    present = cnt > jnp.float32(0.0)
    initd = init_ref[...] > 0
    out_ref[...] = jnp.where(present, jnp.where(initd, ema, mean), protos)


_phase_b = pl.pallas_call(
    _phase_b_body,
    out_shape=jax.ShapeDtypeStruct((CPAD, D), jnp.float32),
)


def kernel(features, labels, prototypes, proto_initialized):
    sums, counts = _phase_a(features, labels)
    protos_pad = jnp.pad(prototypes, ((0, CPAD - C), (0, 0)))
    init_pad = jnp.pad(proto_initialized.astype(jnp.int32),
                       (0, CPAD - C)).reshape(CPAD, 1)
    new_protos = _phase_b(sums, counts, protos_pad, init_pad)
    return new_protos[:C]


# speedup vs baseline: 1.2050x; 1.2050x over previous
import jax
import jax.numpy as jnp
from jax import lax
from jax.experimental import pallas as pl
from jax.experimental.pallas import tpu as pltpu
from jax.experimental.pallas import tpu_sc as plsc

N = 16384
D = 256
C = 1000
CPAD = 1024
NC = 2
NS = 16
LANES = 16
NG = 8
NK = 4
KCOLS = D // NK
KV = KCOLS // LANES
ROWS_PER_GROUP = N // NG
CH = 512
NCHUNK = ROWS_PER_GROUP // CH
NBLK = 8
RB = N // NBLK
CB = CPAD // NBLK


def _norm_body(f_ref, o_ref):
    x = f_ref[...]
    ss = jnp.sum(x * x, axis=1)
    norm = jnp.sqrt(ss)
    o_ref[...] = (jnp.float32(1.0) / jnp.maximum(norm, jnp.float32(1e-12)))[None, None, :]


_norms = pl.pallas_call(
    _norm_body,
    grid=(NBLK,),
    in_specs=[pl.BlockSpec((RB, D), lambda i: (i, 0))],
    out_specs=pl.BlockSpec((1, 1, RB), lambda i: (i, 0, 0)),
    out_shape=jax.ShapeDtypeStruct((NBLK, 1, RB), jnp.float32),
)


def _scatter_body(features, labels, invn, parts_out, cnts_out,
                  featchunk, labelbuf, invnbuf, acc, cnt):
    c = lax.axis_index("c")
    s = lax.axis_index("s")
    g = c * 4 + s // 4
    k = s % 4
    wp = k * NG + g

    zeros16 = jnp.zeros((LANES,), jnp.float32)
    ones16 = jnp.ones((LANES,), jnp.float32)

    def _zrow(r, carry):
        for j in range(KV):
            acc[r, pl.ds(j * LANES, LANES)] = zeros16
        cnt[r, :] = zeros16
        return carry
    lax.fori_loop(0, CPAD, _zrow, 0)

    col0 = k * KCOLS
    for m in range(NCHUNK):
        row0 = g * ROWS_PER_GROUP + m * CH
        pltpu.sync_copy(features.at[pl.ds(row0, CH), pl.ds(col0, KCOLS)],
                        featchunk)
        pltpu.sync_copy(labels.at[pl.ds(row0, CH)], labelbuf)
        pltpu.sync_copy(invn.at[pl.ds(row0, CH)], invnbuf)

        def _acc16(t, carry):
            labs = labelbuf[pl.ds(t * LANES, LANES)]
            ws = invnbuf[pl.ds(t * LANES, LANES)]
            for i in range(LANES):
                lab = labs[i]
                w = jnp.broadcast_to(ws[i], (LANES,))
                r = t * LANES + i
                for j in range(KV):
                    acc[lab, pl.ds(j * LANES, LANES)] = (
                        acc[lab, pl.ds(j * LANES, LANES)]
                        + featchunk[r, pl.ds(j * LANES, LANES)] * w)
            return carry
        lax.fori_loop(0, CH // LANES, _acc16, 0)

        @pl.when(k == m % NK)
        def _():
            def _cnt16(t, carry):
                labs = labelbuf[pl.ds(t * LANES, LANES)]
                for i in range(LANES):
                    lab = labs[i]
                    cnt[lab, :] = cnt[lab, :] + ones16
                return carry
            lax.fori_loop(0, CH // LANES, _cnt16, 0)

    pltpu.sync_copy(acc, parts_out.at[wp])
    pltpu.sync_copy(cnt, cnts_out.at[wp])


_scatter = pl.kernel(
    _scatter_body,
    out_type=[
        jax.ShapeDtypeStruct((NK * NG, CPAD, KCOLS), jnp.float32),
        jax.ShapeDtypeStruct((NK * NG, CPAD, LANES), jnp.float32),
    ],
    mesh=plsc.VectorSubcoreMesh(
        core_axis_name="c", subcore_axis_name="s",
        num_cores=NC, num_subcores=NS),
    compiler_params=pltpu.CompilerParams(
        needs_layout_passes=False, use_tc_tiling_on_sc=False),
    scratch_types=[
        pltpu.VMEM((CH, KCOLS), jnp.float32),
        pltpu.VMEM((CH,), jnp.int32),
        pltpu.VMEM((CH,), jnp.float32),
        pltpu.VMEM((CPAD, KCOLS), jnp.float32),
        pltpu.VMEM((CPAD, LANES), jnp.float32),
    ],
)


def _epilogue_body(parts_ref, cnts_ref, protos_ref, init_ref, out_ref):
    p = parts_ref[...]
    p = p.reshape(NK, NG, CB, KCOLS)
    s_k = jnp.sum(p, axis=1)
    sums = jnp.concatenate([s_k[kk] for kk in range(NK)], axis=-1)
    cnt = jnp.sum(cnts_ref[...], axis=0)[:, 0:1]
    mean = sums / jnp.maximum(cnt, jnp.float32(1.0))
    protos = protos_ref[...]
    ema = jnp.float32(0.99) * protos + jnp.float32(0.01) * mean
    present = cnt > jnp.float32(0.0)
    initd = init_ref[...] > 0
    out_ref[...] = jnp.where(present, jnp.where(initd, ema, mean), protos)


_epilogue = pl.pallas_call(
    _epilogue_body,
    grid=(NBLK,),
    in_specs=[
        pl.BlockSpec((NK * NG, CB, KCOLS), lambda i: (0, i, 0)),
        pl.BlockSpec((NK * NG, CB, LANES), lambda i: (0, i, 0)),
        pl.BlockSpec((CB, D), lambda i: (i, 0)),
        pl.BlockSpec((CB, 1), lambda i: (i, 0)),
    ],
    out_specs=pl.BlockSpec((CB, D), lambda i: (i, 0)),
    out_shape=jax.ShapeDtypeStruct((CPAD, D), jnp.float32),
)


def kernel(features, labels, prototypes, proto_initialized):
    invn = _norms(features).reshape(N)
    parts, cnts = _scatter(features, labels, invn)
    protos_pad = jnp.pad(prototypes, ((0, CPAD - C), (0, 0)))
    init_pad = jnp.pad(proto_initialized.astype(jnp.int32),
                       (0, CPAD - C)).reshape(CPAD, 1)
    new_protos = _epilogue(parts, cnts, protos_pad, init_pad)
    return new_protos[:C]


# speedup vs baseline: 1.3166x; 1.0926x over previous
import jax
import jax.numpy as jnp
from jax import lax
from jax.experimental import pallas as pl
from jax.experimental.pallas import tpu as pltpu
from jax.experimental.pallas import tpu_sc as plsc

N = 16384
D = 256
C = 1000
CPAD = 1024
NC = 2
NS = 16
LANES = 16
NG = 8
NK = 4
KCOLS = D // NK
KV = KCOLS // LANES
ROWS_PER_GROUP = N // NG
CH = 512
NCHUNK = ROWS_PER_GROUP // CH
NBLK = 8
RB = N // NBLK
CB = CPAD // NBLK


def _norm_body(f_ref, o_ref):
    x = f_ref[...]
    ss = jnp.sum(x * x, axis=1)
    norm = jnp.sqrt(ss)
    o_ref[...] = (jnp.float32(1.0) / jnp.maximum(norm, jnp.float32(1e-12)))[None, None, :]


_norms = pl.pallas_call(
    _norm_body,
    grid=(NBLK,),
    in_specs=[pl.BlockSpec((RB, D), lambda i: (i, 0))],
    out_specs=pl.BlockSpec((1, 1, RB), lambda i: (i, 0, 0)),
    out_shape=jax.ShapeDtypeStruct((NBLK, 1, RB), jnp.float32),
)


def _scatter_body(features, labels, invn, parts_out, cnts_out,
                  featchunk, labelbuf, invnbuf, acc, cnt):
    c = lax.axis_index("c")
    s = lax.axis_index("s")
    g = c * 4 + s // 4
    k = s % 4
    wp = k * NG + g

    zeros16 = jnp.zeros((LANES,), jnp.float32)
    ones16 = jnp.ones((LANES,), jnp.float32)

    def _zrow(r, carry):
        for j in range(KV):
            acc[r, pl.ds(j * LANES, LANES)] = zeros16
        cnt[r, :] = zeros16
        return carry
    lax.fori_loop(0, CPAD, _zrow, 0)

    col0 = k * KCOLS
    for m in range(NCHUNK):
        row0 = g * ROWS_PER_GROUP + m * CH
        pltpu.sync_copy(features.at[pl.ds(row0, CH), pl.ds(col0, KCOLS)],
                        featchunk)
        pltpu.sync_copy(labels.at[pl.ds(row0, CH)], labelbuf)
        pltpu.sync_copy(invn.at[pl.ds(row0, CH)], invnbuf)

        def _acc16(t, carry):
            labs = labelbuf[pl.ds(t * LANES, LANES)]
            ws = invnbuf[pl.ds(t * LANES, LANES)]
            for i in range(LANES):
                lab = labs[i]
                w = jnp.broadcast_to(ws[i], (LANES,))
                r = t * LANES + i
                for j in range(KV):
                    plsc.addupdate(
                        acc.at[lab, pl.ds(j * LANES, LANES)],
                        featchunk[r, pl.ds(j * LANES, LANES)] * w)
            return carry
        lax.fori_loop(0, CH // LANES, _acc16, 0)

        @pl.when(k == m % NK)
        def _():
            def _cnt16(t, carry):
                labs = labelbuf[pl.ds(t * LANES, LANES)]
                for i in range(LANES):
                    lab = labs[i]
                    plsc.addupdate(cnt.at[lab, :], ones16)
                return carry
            lax.fori_loop(0, CH // LANES, _cnt16, 0)

    pltpu.sync_copy(acc, parts_out.at[wp])
    pltpu.sync_copy(cnt, cnts_out.at[wp])


_scatter = pl.kernel(
    _scatter_body,
    out_type=[
        jax.ShapeDtypeStruct((NK * NG, CPAD, KCOLS), jnp.float32),
        jax.ShapeDtypeStruct((NK * NG, CPAD, LANES), jnp.float32),
    ],
    mesh=plsc.VectorSubcoreMesh(
        core_axis_name="c", subcore_axis_name="s",
        num_cores=NC, num_subcores=NS),
    compiler_params=pltpu.CompilerParams(
        needs_layout_passes=False, use_tc_tiling_on_sc=False),
    scratch_types=[
        pltpu.VMEM((CH, KCOLS), jnp.float32),
        pltpu.VMEM((CH,), jnp.int32),
        pltpu.VMEM((CH,), jnp.float32),
        pltpu.VMEM((CPAD, KCOLS), jnp.float32),
        pltpu.VMEM((CPAD, LANES), jnp.float32),
    ],
)


def _epilogue_body(parts_ref, cnts_ref, protos_ref, init_ref, out_ref):
    p = parts_ref[...]
    p = p.reshape(NK, NG, CB, KCOLS)
    s_k = jnp.sum(p, axis=1)
    sums = jnp.concatenate([s_k[kk] for kk in range(NK)], axis=-1)
    cnt = jnp.sum(cnts_ref[...], axis=0)[:, 0:1]
    mean = sums / jnp.maximum(cnt, jnp.float32(1.0))
    protos = protos_ref[...]
    ema = jnp.float32(0.99) * protos + jnp.float32(0.01) * mean
    present = cnt > jnp.float32(0.0)
    initd = init_ref[...] > 0
    out_ref[...] = jnp.where(present, jnp.where(initd, ema, mean), protos)


_epilogue = pl.pallas_call(
    _epilogue_body,
    grid=(NBLK,),
    in_specs=[
        pl.BlockSpec((NK * NG, CB, KCOLS), lambda i: (0, i, 0)),
        pl.BlockSpec((NK * NG, CB, LANES), lambda i: (0, i, 0)),
        pl.BlockSpec((CB, D), lambda i: (i, 0)),
        pl.BlockSpec((CB, 1), lambda i: (i, 0)),
    ],
    out_specs=pl.BlockSpec((CB, D), lambda i: (i, 0)),
    out_shape=jax.ShapeDtypeStruct((CPAD, D), jnp.float32),
)


def kernel(features, labels, prototypes, proto_initialized):
    invn = _norms(features).reshape(N)
    parts, cnts = _scatter(features, labels, invn)
    protos_pad = jnp.pad(prototypes, ((0, CPAD - C), (0, 0)))
    init_pad = jnp.pad(proto_initialized.astype(jnp.int32),
                       (0, CPAD - C)).reshape(CPAD, 1)
    new_protos = _epilogue(parts, cnts, protos_pad, init_pad)
    return new_protos[:C]


# speedup vs baseline: 1.5659x; 1.1894x over previous
import jax
import jax.numpy as jnp
from jax import lax
from jax.experimental import pallas as pl
from jax.experimental.pallas import tpu as pltpu
from jax.experimental.pallas import tpu_sc as plsc

N = 16384
D = 256
C = 1000
CPAD = 1024
NC = 2
NS = 16
LANES = 16
NG = 8
NK = 4
KCOLS = D // NK
KV = KCOLS // LANES
ROWS_PER_GROUP = N // NG
CH = 512
NCHUNK = ROWS_PER_GROUP // CH
NBLK = 8
RB = N // NBLK
CB = CPAD // NBLK


def _norm_body(f_ref, o_ref):
    x = f_ref[...]
    ss = jnp.sum(x * x, axis=1)
    norm = jnp.sqrt(ss)
    w = (jnp.float32(1.0) / jnp.maximum(norm, jnp.float32(1e-12)))[:, None]
    xn = x * w
    o_ref[...] = jnp.stack([xn[:, :128], xn[:, 128:]], axis=0)


_norms = pl.pallas_call(
    _norm_body,
    grid=(NBLK,),
    in_specs=[pl.BlockSpec((RB, D), lambda i: (i, 0))],
    out_specs=pl.BlockSpec((2, RB, 128), lambda i: (0, i, 0)),
    out_shape=jax.ShapeDtypeStruct((2, N, 128), jnp.float32),
)


def _scatter_body(featn, labels, parts_out, cnts_out,
                  featchunk, labelbuf, acc, cnt):
    c = lax.axis_index("c")
    s = lax.axis_index("s")
    g = c * 4 + s // 4
    k = s % 4
    wp = k * NG + g

    zeros16 = jnp.zeros((LANES,), jnp.float32)
    ones16 = jnp.ones((LANES,), jnp.float32)

    def _zrow(r, carry):
        for j in range(KV):
            acc[r, pl.ds(j * LANES, LANES)] = zeros16
        cnt[r, :] = zeros16
        return carry
    lax.fori_loop(0, CPAD, _zrow, 0)

    plane = k // 2
    col0 = (k % 2) * KCOLS
    for m in range(NCHUNK):
        row0 = g * ROWS_PER_GROUP + m * CH
        pltpu.sync_copy(featn.at[plane, pl.ds(row0, CH), pl.ds(col0, KCOLS)],
                        featchunk)
        pltpu.sync_copy(labels.at[pl.ds(row0, CH)], labelbuf)

        def _acc16(t, carry):
            labs = labelbuf[pl.ds(t * LANES, LANES)]
            for i in range(LANES):
                lab = labs[i]
                r = t * LANES + i
                for j in range(KV):
                    plsc.addupdate(
                        acc.at[lab, pl.ds(j * LANES, LANES)],
                        featchunk[r, pl.ds(j * LANES, LANES)])
            return carry
        lax.fori_loop(0, CH // LANES, _acc16, 0)

        @pl.when(k == m % NK)
        def _():
            def _cnt16(t, carry):
                labs = labelbuf[pl.ds(t * LANES, LANES)]
                for i in range(LANES):
                    lab = labs[i]
                    plsc.addupdate(cnt.at[lab, :], ones16)
                return carry
            lax.fori_loop(0, CH // LANES, _cnt16, 0)

    pltpu.sync_copy(acc, parts_out.at[wp])
    pltpu.sync_copy(cnt, cnts_out.at[wp])


_scatter = pl.kernel(
    _scatter_body,
    out_type=[
        jax.ShapeDtypeStruct((NK * NG, CPAD, KCOLS), jnp.float32),
        jax.ShapeDtypeStruct((NK * NG, CPAD, LANES), jnp.float32),
    ],
    mesh=plsc.VectorSubcoreMesh(
        core_axis_name="c", subcore_axis_name="s",
        num_cores=NC, num_subcores=NS),
    compiler_params=pltpu.CompilerParams(
        needs_layout_passes=False, use_tc_tiling_on_sc=False),
    scratch_types=[
        pltpu.VMEM((CH, KCOLS), jnp.float32),
        pltpu.VMEM((CH,), jnp.int32),
        pltpu.VMEM((CPAD, KCOLS), jnp.float32),
        pltpu.VMEM((CPAD, LANES), jnp.float32),
    ],
)


def _epilogue_body(parts_ref, cnts_ref, protos_ref, init_ref, out_ref):
    p = parts_ref[...]
    p = p.reshape(NK, NG, CB, KCOLS)
    s_k = jnp.sum(p, axis=1)
    sums = jnp.concatenate([s_k[kk] for kk in range(NK)], axis=-1)
    cnt = jnp.sum(cnts_ref[...], axis=0)[:, 0:1]
    mean = sums / jnp.maximum(cnt, jnp.float32(1.0))
    protos = protos_ref[...]
    ema = jnp.float32(0.99) * protos + jnp.float32(0.01) * mean
    present = cnt > jnp.float32(0.0)
    initd = init_ref[...] > 0
    out_ref[...] = jnp.where(present, jnp.where(initd, ema, mean), protos)


_epilogue = pl.pallas_call(
    _epilogue_body,
    grid=(NBLK,),
    in_specs=[
        pl.BlockSpec((NK * NG, CB, KCOLS), lambda i: (0, i, 0)),
        pl.BlockSpec((NK * NG, CB, LANES), lambda i: (0, i, 0)),
        pl.BlockSpec((CB, D), lambda i: (i, 0)),
        pl.BlockSpec((CB, 1), lambda i: (i, 0)),
    ],
    out_specs=pl.BlockSpec((CB, D), lambda i: (i, 0)),
    out_shape=jax.ShapeDtypeStruct((CPAD, D), jnp.float32),
)


def kernel(features, labels, prototypes, proto_initialized):
    featn = _norms(features)
    parts, cnts = _scatter(featn, labels)
    protos_pad = jnp.pad(prototypes, ((0, CPAD - C), (0, 0)))
    init_pad = jnp.pad(proto_initialized.astype(jnp.int32),
                       (0, CPAD - C)).reshape(CPAD, 1)
    new_protos = _epilogue(parts, cnts, protos_pad, init_pad)
    return new_protos[:C]
